# Initial kernel scaffold; baseline (speedup 1.0000x reference)
import jax
import jax.numpy as jnp
from jax.experimental import pallas as pl


def kernel(input_ids, embed_table):
    raise NotImplementedError("write your pallas kernel here")


# speedup vs baseline: 1.7655x; 1.7655x over previous
import functools

import jax
import jax.numpy as jnp
from jax import lax
from jax.experimental import pallas as pl
from jax.experimental.pallas import tpu as pltpu
from jax.experimental.pallas import tpu_sc as plsc

NC, NS = 2, 16
NW = NC * NS
K = 16


def _make_gather(n_ids: int, hidden: int):
  bpw = n_ids // NW
  nch = bpw // K
  mesh = plsc.VectorSubcoreMesh(core_axis_name="c", subcore_axis_name="s")

  @functools.partial(
      pl.kernel,
      mesh=mesh,
      out_type=jax.ShapeDtypeStruct((n_ids, hidden), jnp.float32),
      scratch_types=[
          pltpu.VMEM((bpw,), jnp.int32),
          pltpu.VMEM((K, hidden), jnp.float32),
          pltpu.VMEM((K, hidden), jnp.float32),
          pltpu.SemaphoreType.DMA,
          pltpu.SemaphoreType.DMA,
      ],
  )
  def gather(tbl_hbm, idx_hbm, out_hbm, idx_v, b0, b1, s0, s1):
    wid = lax.axis_index("s") * NC + lax.axis_index("c")
    base = pl.multiple_of(wid * bpw, 8)
    pltpu.sync_copy(idx_hbm.at[pl.ds(base, bpw)], idx_v)

    def start(ch, buf, sem):
      off = pl.multiple_of(ch * K, 8)
      return pltpu.async_copy(tbl_hbm.at[idx_v.at[pl.ds(off, K)]], buf, sem)

    def finish(ch, buf, sem):
      off = pl.multiple_of(ch * K, 8)
      pltpu.make_async_copy(
          tbl_hbm.at[idx_v.at[pl.ds(off, K)]], buf, sem).wait()
      pltpu.sync_copy(buf, out_hbm.at[pl.ds(base + off, K)])

    start(0, b0, s0)
    start(1, b1, s1)

    @pl.loop(0, nch, step=2)
    def _(t):
      finish(t, b0, s0)

      @pl.when(t + 2 < nch)
      def _():
        start(t + 2, b0, s0)

      finish(t + 1, b1, s1)

      @pl.when(t + 3 < nch)
      def _():
        start(t + 3, b1, s1)

  return gather


@jax.jit
def kernel(input_ids, embed_table):
  b, s = input_ids.shape
  v, h = embed_table.shape
  ids = input_ids.reshape(b * s).astype(jnp.int32)
  out = _make_gather(b * s, h)(embed_table, ids)
  return out.reshape(b, s, h)


# speedup vs baseline: 1.7843x; 1.0107x over previous
import functools

import jax
import jax.numpy as jnp
from jax import lax
from jax.experimental import pallas as pl
from jax.experimental.pallas import tpu as pltpu
from jax.experimental.pallas import tpu_sc as plsc

NC, NS = 2, 16
NW = NC * NS
K = 8
NBUF = 4
LAG = 2


def _make_gather(n_ids: int, hidden: int):
  bpw = n_ids // NW
  nch = bpw // K
  mesh = plsc.VectorSubcoreMesh(core_axis_name="c", subcore_axis_name="s")

  @functools.partial(
      pl.kernel,
      mesh=mesh,
      out_type=jax.ShapeDtypeStruct((n_ids, hidden), jnp.float32),
      scratch_types=[
          pltpu.VMEM((bpw,), jnp.int32),
          [pltpu.VMEM((K, hidden), jnp.float32) for _ in range(NBUF)],
          [pltpu.SemaphoreType.DMA for _ in range(NBUF)],
          [pltpu.SemaphoreType.DMA for _ in range(NBUF)],
      ],
  )
  def gather(tbl_hbm, idx_hbm, out_hbm, idx_v, bufs, gsems, ssems):
    wid = lax.axis_index("s") * NC + lax.axis_index("c")
    base = pl.multiple_of(wid * bpw, 8)
    pltpu.sync_copy(idx_hbm.at[pl.ds(base, bpw)], idx_v)

    def g_desc(ch, j):
      off = pl.multiple_of(ch * K, 8)
      return pltpu.make_async_copy(
          tbl_hbm.at[idx_v.at[pl.ds(off, K)]], bufs[j], gsems[j])

    def s_desc(ch, j):
      off = pl.multiple_of(ch * K, 8)
      return pltpu.make_async_copy(
          bufs[j], out_hbm.at[pl.ds(base + off, K)], ssems[j])

    for j in range(NBUF - LAG):
      g_desc(j, j).start()

    @pl.loop(0, nch, step=NBUF)
    def _(t):
      for j in range(NBUF):
        ch = t + j
        g_desc(ch, j).wait()
        s_desc(ch, j).start()
        jn = (j + NBUF - LAG) % NBUF

        @pl.when(ch + NBUF - LAG < nch)
        def _():
          @pl.when(ch + NBUF - LAG >= NBUF)
          def _():
            s_desc(ch - LAG, jn).wait()
          g_desc(ch + NBUF - LAG, jn).start()

    for c in range(nch - NBUF, nch):
      s_desc(c, c % NBUF).wait()

  return gather


@jax.jit
def kernel(input_ids, embed_table):
  b, s = input_ids.shape
  v, h = embed_table.shape
  ids = input_ids.reshape(b * s).astype(jnp.int32)
  out = _make_gather(b * s, h)(embed_table, ids)
  return out.reshape(b, s, h)
